# baseline (speedup 1.0000x reference)
import jax
import jax.numpy as jnp
from jax.experimental import pallas as pl
from jax.experimental.pallas import tpu as pltpu
from jax.experimental.pallas import tpu_sc as plsc

B = 1024
D_C = 48
N_CAT = 16
CAT_DIM = 10
N_ORD = 16
D = D_C + N_CAT + N_ORD
HD = 64
TED = 64
NL = 3
NT = 3
E = 1280
CIN = 2 * HD + TED

F32 = jnp.float32
BF16 = jnp.bfloat16


def _dot(a, b, dims=((1,), (0,))):
    return jax.lax.dot_general(a, b, (dims, ((), ())),
                               preferred_element_type=F32)


def _gelu(z):
    return 0.5 * z * (1.0 + jax.lax.erf(z * 0.7071067811865476))


def _full(s):
    return pl.BlockSpec(s, lambda *_: (0,) * len(s))


NSC = 2
DHALF = D // NSC

_scalar_mesh = plsc.ScalarSubcoreMesh(axis_name="core", num_cores=NSC)


def _adj_sc_kernel(ei_hbm, ew_hbm, a_hbm, deg_hbm, eib, ewb, arow, degrow,
                   sem):
    c = jax.lax.axis_index("core")
    pltpu.async_copy(ei_hbm, eib, sem).wait()
    pltpu.async_copy(ew_hbm, ewb, sem).wait()

    @pl.loop(0, DHALF * D)
    def _(i):
        arow[i] = 0.0

    @pl.loop(0, 128)
    def _(i):
        degrow[i] = 0.0

    lo = c * DHALF

    @pl.loop(0, E)
    def _(e):
        d = eib[E + e]

        @pl.when(jnp.logical_and(d >= lo, d < lo + DHALF))
        def _():
            s = eib[e]
            w = ewb[e]
            arow[(d - lo) * D + s] += w
            degrow[d - lo] += w

    pltpu.async_copy(arow, a_hbm.at[pl.ds(c * DHALF * D, DHALF * D)],
                     sem).wait()
    pltpu.async_copy(degrow, deg_hbm.at[pl.ds(c * 128, 128)],
                     sem).wait()


def _adj_sc(edge_index, edge_weight):
    @pl.kernel(
        out_type=[
            jax.ShapeDtypeStruct((D * D,), F32),
            jax.ShapeDtypeStruct((NSC * 128,), F32),
        ],
        mesh=_scalar_mesh,
        scratch_types=[
            pltpu.SMEM((2 * E,), jnp.int32),
            pltpu.SMEM((E,), F32),
            pltpu.SMEM((DHALF * D,), F32),
            pltpu.SMEM((128,), F32),
            pltpu.SemaphoreType.DMA,
        ],
    )
    def run(ei_hbm, ew_hbm, a_hbm, deg_hbm, eib, ewb, arow, degrow, sem):
        _adj_sc_kernel(ei_hbm, ew_hbm, a_hbm, deg_hbm, eib, ewb, arow,
                       degrow, sem)

    a_raw, deg_pad = run(edge_index.reshape(2 * E), edge_weight)
    deg = deg_pad.reshape(NSC, 128)[:, :DHALF].reshape(D, 1)
    return a_raw.reshape(D, D), deg


BB_P = 256


def _proj_kernel(xcT_ref, xd_ref, xo_ref, wc_ref, bc_ref,
                 Wcat_ref, bcat_ref, WordT_ref, bord_ref, h_ref):
    h_ref[0:D_C] = xcT_ref[...] * wc_ref[...] + bc_ref[...]
    for k in range(N_CAT):
        yk = _dot(xd_ref[k], Wcat_ref[k])
        h_ref[D_C + k] = yk + bcat_ref[k:k + 1, :]
    xo = xo_ref[...]
    c = jnp.cos(xo)
    s = jnp.sin(xo)
    WT = WordT_ref[...]
    for k in range(N_ORD):
        cs = jnp.concatenate([c[k:k + 1, :], s[k:k + 1, :]], axis=0)
        ek = _dot(WT, cs)
        h_ref[D_C + N_CAT + k] = ek.T + bord_ref[...]


BB_L = 128


def _layer_update(a_ref, deg_ref, hb, t_ref, nt_ref, W_ref, b_ref):
    an = a_ref[...] / jnp.maximum(deg_ref[...], 1e-8)
    agg = _dot(an, hb)
    h2 = hb.reshape(D * BB_L, HD)
    a2 = agg.reshape(D * BB_L, HD)
    W = W_ref[...]
    p_all = _dot(h2, W[0:HD]) + _dot(a2, W[HD:2 * HD])
    tc_all = _dot(t_ref[...], W[2 * HD:CIN]) + b_ref[...]
    nt = nt_ref[...]
    p_sel = jnp.zeros((D, BB_L, HD), F32)
    for t in range(NT):
        p = p_all[:, t * HD:(t + 1) * HD].reshape(D, BB_L, HD)
        tc = tc_all[:, t * HD:(t + 1) * HD]
        mask = (nt == t).astype(F32)
        p_sel = p_sel + (p + tc[None]) * mask
    return hb + _gelu(p_sel)


def _layer_kernel(a_ref, deg_ref, h_ref, t_ref, nt_ref, W_ref, b_ref,
                  out_ref):
    out_ref[...] = _layer_update(a_ref, deg_ref, h_ref[...], t_ref, nt_ref,
                                 W_ref, b_ref)


def _last_kernel(a_ref, deg_ref, h_ref, t_ref, nt_ref, W_ref, b_ref,
                 woc_ref, boc_ref, Wocat_ref, bocat_ref, wood_ref, bood_ref,
                 vcT_ref, vd_ref, vo_ref):
    h = _layer_update(a_ref, deg_ref, h_ref[...], t_ref, nt_ref, W_ref,
                      b_ref)
    vcT = jnp.sum(h[0:D_C] * woc_ref[...], axis=-1)
    vcT_ref[...] = vcT + boc_ref[...]
    for k in range(N_CAT):
        yk = _dot(h[D_C + k], Wocat_ref[k])
        yk = yk + bocat_ref[k:k + 1, :]
        vd_ref[k] = yk - jnp.mean(yk, axis=-1, keepdims=True)
    vo = jnp.sum(h[D_C + N_CAT:D] * wood_ref[...], axis=-1)
    vo_ref[...] = vo + bood_ref[...]


def kernel(x_c, x_d_list, x_o_list, t_emb, edge_index, edge_weight,
           node_types, wc, bc, Wcat, bcat, Word, bord, Wup, bup,
           woc, boc, Wocat, bocat, wood, bood):
    xcT = x_c.T[:, :, None]
    ew2 = edge_weight[None, :]
    nt3 = node_types[:, None, None]
    wc3 = wc[None, None, :]
    bc3 = bc[None, None, :]
    bord2 = bord[None, :]
    woc3 = woc[None, None, :]
    wood3 = wood[None, None, :]
    boc2 = boc[None, :]
    bood2 = bood[None, :]

    A_raw, deg = _adj_sc(edge_index, edge_weight)

    h = pl.pallas_call(
        _proj_kernel,
        grid=(B // BB_P,),
        in_specs=[
            pl.BlockSpec((D_C, BB_P, 1), lambda i: (0, i, 0)),
            pl.BlockSpec((N_CAT, BB_P, CAT_DIM), lambda i: (0, i, 0)),
            pl.BlockSpec((N_ORD, BB_P), lambda i: (0, i)),
            _full((1, 1, HD)), _full((1, 1, HD)),
            _full((N_CAT, CAT_DIM, HD)), _full((N_CAT, HD)),
            _full((HD, 2)), _full((1, HD)),
        ],
        out_specs=pl.BlockSpec((D, BB_P, HD), lambda i: (0, i, 0)),
        out_shape=jax.ShapeDtypeStruct((D, B, HD), F32),
    )(xcT, x_d_list, x_o_list, wc3, bc3, Wcat, bcat, Word.T, bord2)

    Wpack = jnp.transpose(Wup, (0, 2, 1, 3)).reshape(NL, CIN, NT * HD)
    bpack = bup.reshape(NL, 1, NT * HD)

    layer_in_specs = [
        _full((D, D)), _full((D, 1)),
        pl.BlockSpec((D, BB_L, HD), lambda i: (0, i, 0)),
        pl.BlockSpec((BB_L, TED), lambda i: (i, 0)),
        _full((D, 1, 1)),
        _full((CIN, NT * HD)), _full((1, NT * HD)),
    ]
    layer_call = pl.pallas_call(
        _layer_kernel,
        grid=(B // BB_L,),
        in_specs=layer_in_specs,
        out_specs=pl.BlockSpec((D, BB_L, HD), lambda i: (0, i, 0)),
        out_shape=jax.ShapeDtypeStruct((D, B, HD), F32),
    )
    for l in range(NL - 1):
        h = layer_call(A_raw, deg, h, t_emb, nt3, Wpack[l], bpack[l])

    vcT, v_d, v_o = pl.pallas_call(
        _last_kernel,
        grid=(B // BB_L,),
        in_specs=layer_in_specs + [
            _full((1, 1, HD)), _full((1, 1)),
            _full((N_CAT, HD, CAT_DIM)), _full((N_CAT, CAT_DIM)),
            _full((1, 1, HD)), _full((1, 1)),
        ],
        out_specs=[
            pl.BlockSpec((D_C, BB_L), lambda i: (0, i)),
            pl.BlockSpec((N_CAT, BB_L, CAT_DIM), lambda i: (0, i, 0)),
            pl.BlockSpec((N_ORD, BB_L), lambda i: (0, i)),
        ],
        out_shape=[
            jax.ShapeDtypeStruct((D_C, B), F32),
            jax.ShapeDtypeStruct((N_CAT, B, CAT_DIM), F32),
            jax.ShapeDtypeStruct((N_ORD, B), F32),
        ],
    )(A_raw, deg, h, t_emb, nt3, Wpack[NL - 1], bpack[NL - 1],
      woc3, boc2, Wocat, bocat, wood3, bood2)
    return vcT.T, v_d, v_o
